# baseline (speedup 1.0000x reference)
import jax
import jax.numpy as jnp
from jax.experimental import pallas as pl

N = 4096
D = 64
OUT = 256
K = 20
ALPHA = 3.0
BT = 16
ROWS = 512


def _dg(a, b, contract):
    return jax.lax.dot_general(
        a, b, (((contract[0],), (contract[1],)), ((), ())),
        preferred_element_type=jnp.float32)


def _preamble_body(emb1_ref, lin1_ref, emb2_ref, lin2_ref, truth_ref,
                   wd_ref, bd_ref, wg_ref, wc_ref,
                   nv1_ref, nv2_ref, yext_ref):
    nv1_ref[...] = jnp.tanh(ALPHA * _dg(emb1_ref[...], lin1_ref[...], (1, 0)))
    nv2_ref[...] = jnp.tanh(ALPHA * _dg(emb2_ref[...], lin2_ref[...], (1, 0)))
    u = _dg(wg_ref[...], wc_ref[...], (1, 0))
    truth = truth_ref[...]
    acc = jnp.zeros((BT, N), jnp.float32)
    for d in range(D):
        wd = wd_ref[0:1, d:d + 1]
        bd = bd_ref[0:1, d:d + 1]
        ud = u[d:d + 1, 0:1]
        acc = acc + ud * jnp.tanh(truth * wd + bd)
    yext_ref[0:BT, :] = acc
    yext_ref[BT:BT + 1, :] = jnp.ones((1, N), jnp.float32)


def _main_body(nv1t_ref, nv2t_ref, nv1f_ref, nv2f_ref, yext_ref,
               tr_ref, mk_ref, wm_ref, bg_ref, wct_ref, bc_ref, bm_ref,
               out_ref):
    i = pl.program_id(0)
    nv1t = nv1t_ref[...]
    nv2t = nv2t_ref[...]
    raw = _dg(nv1t, nv2f_ref[...], (1, 1)) - _dg(nv2t, nv1f_ref[...], (1, 1))
    a = jnp.maximum(jnp.tanh(ALPHA * raw), 0.0)
    t = jnp.full((ROWS, 1), jnp.inf, jnp.float32)
    for _ in range(K):
        c = a >= t
        cnt = jnp.sum(jnp.where(c, 1.0, 0.0), axis=1, keepdims=True)
        newt = jnp.max(jnp.where(c, -1.0, a), axis=1, keepdims=True)
        t = jnp.where(cnt >= K, t, newt)
    gt = a > t
    g = jnp.sum(jnp.where(gt, 1.0, 0.0), axis=1, keepdims=True)
    eq = a == t
    ps = jnp.where(eq, 1.0, 0.0)
    eqf = ps
    sh = 1
    while sh < N:
        ps = ps + jnp.concatenate(
            [jnp.zeros((ROWS, sh), jnp.float32), ps[:, :N - sh]], axis=1)
        sh *= 2
    rank_excl = ps - eqf
    keep = gt | (eq & (rank_excl < (K - g)))
    adjm = jnp.where(keep, a, 0.0)
    z_ext = _dg(yext_ref[...], adjm, (1, 1))
    znorm = z_ext[0:BT, :] / (z_ext[BT:BT + 1, :] + 1e-6)
    c = jnp.sum(bg_ref[...] * wct_ref[...], keepdims=True) + bc_ref[...]
    x2 = jnp.tanh(znorm + c)
    mk = mk_ref[...]
    x2 = tr_ref[...] * mk + x2 * (1.0 - mk)
    contrib = _dg(x2, wm_ref[...], (1, 0))

    @pl.when(i == 0)
    def _():
        out_ref[...] = contrib + bm_ref[...]

    @pl.when(i != 0)
    def _():
        out_ref[...] = out_ref[...] + contrib


def kernel(truth, mask, emb1, emb2, lin1, lin2, W_d, b_d, W_g, b_g, W_c, b_c,
           W_m, b_m):
    B, T, _ = truth.shape
    truth2 = truth.reshape(BT, N)
    mask2 = mask.reshape(BT, N)

    nv1, nv2, yext = pl.pallas_call(
        _preamble_body,
        out_shape=[
            jax.ShapeDtypeStruct((N, D), jnp.float32),
            jax.ShapeDtypeStruct((N, D), jnp.float32),
            jax.ShapeDtypeStruct((BT + 1, N), jnp.float32),
        ],
    )(emb1, lin1, emb2, lin2, truth2,
      W_d.reshape(1, D), b_d.reshape(1, D), W_g, W_c)

    grid = (N // ROWS,)
    out = pl.pallas_call(
        _main_body,
        grid=grid,
        in_specs=[
            pl.BlockSpec((ROWS, D), lambda i: (i, 0)),
            pl.BlockSpec((ROWS, D), lambda i: (i, 0)),
            pl.BlockSpec((N, D), lambda i: (0, 0)),
            pl.BlockSpec((N, D), lambda i: (0, 0)),
            pl.BlockSpec((BT + 1, N), lambda i: (0, 0)),
            pl.BlockSpec((BT, ROWS), lambda i: (0, i)),
            pl.BlockSpec((BT, ROWS), lambda i: (0, i)),
            pl.BlockSpec((ROWS, OUT), lambda i: (i, 0)),
            pl.BlockSpec((1, D), lambda i: (0, 0)),
            pl.BlockSpec((1, D), lambda i: (0, 0)),
            pl.BlockSpec((1, 1), lambda i: (0, 0)),
            pl.BlockSpec((1, OUT), lambda i: (0, 0)),
        ],
        out_specs=pl.BlockSpec((BT, OUT), lambda i: (0, 0)),
        out_shape=jax.ShapeDtypeStruct((BT, OUT), jnp.float32),
    )(nv1, nv2, nv1, nv2, yext, truth2, mask2, W_m,
      b_g.reshape(1, D), W_c.reshape(1, D), b_c.reshape(1, 1),
      b_m.reshape(1, OUT))

    return out.reshape(B, T, OUT)


# speedup vs baseline: 1.2034x; 1.2034x over previous
import jax
import jax.numpy as jnp
from jax.experimental import pallas as pl

N = 4096
D = 64
OUT = 256
K = 20
ALPHA = 3.0
BT = 16
ROWS = 128


def _dg(a, b, contract):
    return jax.lax.dot_general(
        a, b, (((contract[0],), (contract[1],)), ((), ())),
        preferred_element_type=jnp.float32)


def _preamble_body(emb1_ref, lin1_ref, emb2_ref, lin2_ref, truth_ref,
                   wd_ref, bd_ref, wg_ref, wc_ref,
                   nv1_ref, nv2_ref, yext_ref):
    nv1_ref[...] = jnp.tanh(ALPHA * _dg(emb1_ref[...], lin1_ref[...], (1, 0)))
    nv2_ref[...] = jnp.tanh(ALPHA * _dg(emb2_ref[...], lin2_ref[...], (1, 0)))
    u = _dg(wg_ref[...], wc_ref[...], (1, 0))
    truth = truth_ref[...]
    acc = jnp.zeros((BT, N), jnp.float32)
    for d in range(D):
        wd = wd_ref[0:1, d:d + 1]
        bd = bd_ref[0:1, d:d + 1]
        ud = u[d:d + 1, 0:1]
        acc = acc + ud * jnp.tanh(truth * wd + bd)
    yext_ref[0:BT, :] = acc
    yext_ref[BT:BT + 1, :] = jnp.ones((1, N), jnp.float32)


def _main_body(nv1t_ref, nv2t_ref, nv1f_ref, nv2f_ref, yext_ref,
               tr_ref, mk_ref, wm_ref, bg_ref, wct_ref, bc_ref, bm_ref,
               out_ref):
    i = pl.program_id(0)
    nv1t = nv1t_ref[...]
    nv2t = nv2t_ref[...]
    raw = _dg(nv1t, nv2f_ref[...], (1, 1)) - _dg(nv2t, nv1f_ref[...], (1, 1))
    a = jnp.maximum(jnp.tanh(ALPHA * raw), 0.0)
    t = jnp.full((ROWS, 1), jnp.inf, jnp.float32)
    for _ in range(K):
        c = a >= t
        cnt = jnp.sum(jnp.where(c, 1.0, 0.0), axis=1, keepdims=True)
        newt = jnp.max(jnp.where(c, -1.0, a), axis=1, keepdims=True)
        t = jnp.where(cnt >= K, t, newt)
    gt = a > t
    g = jnp.sum(jnp.where(gt, 1.0, 0.0), axis=1, keepdims=True)
    eq = a == t
    ps = jnp.where(eq, 1.0, 0.0)
    eqf = ps
    sh = 1
    while sh < N:
        ps = ps + jnp.concatenate(
            [jnp.zeros((ROWS, sh), jnp.float32), ps[:, :N - sh]], axis=1)
        sh *= 2
    rank_excl = ps - eqf
    keep = gt | (eq & (rank_excl < (K - g)))
    adjm = jnp.where(keep, a, 0.0)
    z_ext = _dg(yext_ref[...], adjm, (1, 1))
    znorm = z_ext[0:BT, :] / (z_ext[BT:BT + 1, :] + 1e-6)
    c = jnp.sum(bg_ref[...] * wct_ref[...], keepdims=True) + bc_ref[...]
    x2 = jnp.tanh(znorm + c)
    mk = mk_ref[...]
    x2 = tr_ref[...] * mk + x2 * (1.0 - mk)
    contrib = _dg(x2, wm_ref[...], (1, 0))

    @pl.when(i == 0)
    def _():
        out_ref[...] = contrib + bm_ref[...]

    @pl.when(i != 0)
    def _():
        out_ref[...] = out_ref[...] + contrib


def kernel(truth, mask, emb1, emb2, lin1, lin2, W_d, b_d, W_g, b_g, W_c, b_c,
           W_m, b_m):
    B, T, _ = truth.shape
    truth2 = truth.reshape(BT, N)
    mask2 = mask.reshape(BT, N)

    nv1, nv2, yext = pl.pallas_call(
        _preamble_body,
        out_shape=[
            jax.ShapeDtypeStruct((N, D), jnp.float32),
            jax.ShapeDtypeStruct((N, D), jnp.float32),
            jax.ShapeDtypeStruct((BT + 1, N), jnp.float32),
        ],
    )(emb1, lin1, emb2, lin2, truth2,
      W_d.reshape(1, D), b_d.reshape(1, D), W_g, W_c)

    grid = (N // ROWS,)
    out = pl.pallas_call(
        _main_body,
        grid=grid,
        in_specs=[
            pl.BlockSpec((ROWS, D), lambda i: (i, 0)),
            pl.BlockSpec((ROWS, D), lambda i: (i, 0)),
            pl.BlockSpec((N, D), lambda i: (0, 0)),
            pl.BlockSpec((N, D), lambda i: (0, 0)),
            pl.BlockSpec((BT + 1, N), lambda i: (0, 0)),
            pl.BlockSpec((BT, ROWS), lambda i: (0, i)),
            pl.BlockSpec((BT, ROWS), lambda i: (0, i)),
            pl.BlockSpec((ROWS, OUT), lambda i: (i, 0)),
            pl.BlockSpec((1, D), lambda i: (0, 0)),
            pl.BlockSpec((1, D), lambda i: (0, 0)),
            pl.BlockSpec((1, 1), lambda i: (0, 0)),
            pl.BlockSpec((1, OUT), lambda i: (0, 0)),
        ],
        out_specs=pl.BlockSpec((BT, OUT), lambda i: (0, 0)),
        out_shape=jax.ShapeDtypeStruct((BT, OUT), jnp.float32),
    )(nv1, nv2, nv1, nv2, yext, truth2, mask2, W_m,
      b_g.reshape(1, D), W_c.reshape(1, D), b_c.reshape(1, 1),
      b_m.reshape(1, OUT))

    return out.reshape(B, T, OUT)


# speedup vs baseline: 1.4608x; 1.2139x over previous
import jax
import jax.numpy as jnp
from jax.experimental import pallas as pl

N = 4096
D = 64
OUT = 256
K = 20
ALPHA = 3.0
BT = 16
ROWS = 256


def _dg(a, b, contract):
    return jax.lax.dot_general(
        a, b, (((contract[0],), (contract[1],)), ((), ())),
        preferred_element_type=jnp.float32)


def _preamble_body(emb1_ref, lin1_ref, emb2_ref, lin2_ref, truth_ref,
                   wd_ref, bd_ref, wg_ref, wc_ref,
                   nv1_ref, nv2_ref, yext_ref):
    nv1_ref[...] = jnp.tanh(ALPHA * _dg(emb1_ref[...], lin1_ref[...], (1, 0)))
    nv2_ref[...] = jnp.tanh(ALPHA * _dg(emb2_ref[...], lin2_ref[...], (1, 0)))
    u = _dg(wg_ref[...], wc_ref[...], (1, 0))
    truth = truth_ref[...]
    acc = jnp.zeros((BT, N), jnp.float32)
    for d in range(D):
        wd = wd_ref[0:1, d:d + 1]
        bd = bd_ref[0:1, d:d + 1]
        ud = u[d:d + 1, 0:1]
        acc = acc + ud * jnp.tanh(truth * wd + bd)
    yext_ref[0:BT, :] = acc
    yext_ref[BT:BT + 1, :] = jnp.ones((1, N), jnp.float32)


def _main_body(nv1t_ref, nv2t_ref, nv1f_ref, nv2f_ref, yext_ref,
               tr_ref, mk_ref, wm_ref, bg_ref, wct_ref, bc_ref, bm_ref,
               out_ref):
    i = pl.program_id(0)
    nv1t = nv1t_ref[...]
    nv2t = nv2t_ref[...]
    raw = _dg(nv1t, nv2f_ref[...], (1, 1)) - _dg(nv2t, nv1f_ref[...], (1, 1))
    a = jnp.maximum(jnp.tanh(ALPHA * raw), 0.0)
    big = 8192.0
    thresh = -(big * K - big / 2 - 2048.0)
    t = jnp.full((ROWS, 1), jnp.inf, jnp.float32)
    for _ in range(K):
        m2 = jnp.where(a >= t, -big, a)
        ssum = jnp.sum(m2, axis=1, keepdims=True)
        newt = jnp.max(m2, axis=1, keepdims=True)
        t = jnp.where(ssum <= thresh, t, newt)
    gt = a > t
    g = jnp.sum(jnp.where(gt, 1.0, 0.0), axis=1, keepdims=True)
    eq = a == t
    ps = jnp.where(eq, 1.0, 0.0)
    eqf = ps
    sh = 1
    while sh < N:
        ps = ps + jnp.concatenate(
            [jnp.zeros((ROWS, sh), jnp.float32), ps[:, :N - sh]], axis=1)
        sh *= 2
    rank_excl = ps - eqf
    keep = gt | (eq & (rank_excl < (K - g)))
    adjm = jnp.where(keep, a, 0.0)
    z_ext = _dg(yext_ref[...], adjm, (1, 1))
    znorm = z_ext[0:BT, :] / (z_ext[BT:BT + 1, :] + 1e-6)
    c = jnp.sum(bg_ref[...] * wct_ref[...], keepdims=True) + bc_ref[...]
    x2 = jnp.tanh(znorm + c)
    mk = mk_ref[...]
    x2 = tr_ref[...] * mk + x2 * (1.0 - mk)
    contrib = _dg(x2, wm_ref[...], (1, 0))

    @pl.when(i == 0)
    def _():
        out_ref[...] = contrib + bm_ref[...]

    @pl.when(i != 0)
    def _():
        out_ref[...] = out_ref[...] + contrib


def kernel(truth, mask, emb1, emb2, lin1, lin2, W_d, b_d, W_g, b_g, W_c, b_c,
           W_m, b_m):
    B, T, _ = truth.shape
    truth2 = truth.reshape(BT, N)
    mask2 = mask.reshape(BT, N)

    nv1, nv2, yext = pl.pallas_call(
        _preamble_body,
        out_shape=[
            jax.ShapeDtypeStruct((N, D), jnp.float32),
            jax.ShapeDtypeStruct((N, D), jnp.float32),
            jax.ShapeDtypeStruct((BT + 1, N), jnp.float32),
        ],
    )(emb1, lin1, emb2, lin2, truth2,
      W_d.reshape(1, D), b_d.reshape(1, D), W_g, W_c)

    grid = (N // ROWS,)
    out = pl.pallas_call(
        _main_body,
        grid=grid,
        in_specs=[
            pl.BlockSpec((ROWS, D), lambda i: (i, 0)),
            pl.BlockSpec((ROWS, D), lambda i: (i, 0)),
            pl.BlockSpec((N, D), lambda i: (0, 0)),
            pl.BlockSpec((N, D), lambda i: (0, 0)),
            pl.BlockSpec((BT + 1, N), lambda i: (0, 0)),
            pl.BlockSpec((BT, ROWS), lambda i: (0, i)),
            pl.BlockSpec((BT, ROWS), lambda i: (0, i)),
            pl.BlockSpec((ROWS, OUT), lambda i: (i, 0)),
            pl.BlockSpec((1, D), lambda i: (0, 0)),
            pl.BlockSpec((1, D), lambda i: (0, 0)),
            pl.BlockSpec((1, 1), lambda i: (0, 0)),
            pl.BlockSpec((1, OUT), lambda i: (0, 0)),
        ],
        out_specs=pl.BlockSpec((BT, OUT), lambda i: (0, 0)),
        out_shape=jax.ShapeDtypeStruct((BT, OUT), jnp.float32),
    )(nv1, nv2, nv1, nv2, yext, truth2, mask2, W_m,
      b_g.reshape(1, D), W_c.reshape(1, D), b_c.reshape(1, 1),
      b_m.reshape(1, OUT))

    return out.reshape(B, T, OUT)


# speedup vs baseline: 1.4739x; 1.0090x over previous
import jax
import jax.numpy as jnp
from jax.experimental import pallas as pl

N = 4096
D = 64
OUT = 256
K = 20
ALPHA = 3.0
BT = 16
ROWS = 256


def _dg(a, b, contract):
    return jax.lax.dot_general(
        a, b, (((contract[0],), (contract[1],)), ((), ())),
        preferred_element_type=jnp.float32)


def _preamble_body(emb1_ref, lin1_ref, emb2_ref, lin2_ref, truth_ref,
                   wd_ref, bd_ref, wg_ref, wc_ref,
                   nv1_ref, nv2_ref, yext_ref):
    nv1_ref[...] = jnp.tanh(ALPHA * _dg(emb1_ref[...], lin1_ref[...], (1, 0)))
    nv2_ref[...] = jnp.tanh(ALPHA * _dg(emb2_ref[...], lin2_ref[...], (1, 0)))
    u = _dg(wg_ref[...], wc_ref[...], (1, 0))
    truth = truth_ref[...]
    acc = jnp.zeros((BT, N), jnp.float32)
    for d in range(D):
        wd = wd_ref[0:1, d:d + 1]
        bd = bd_ref[0:1, d:d + 1]
        ud = u[d:d + 1, 0:1]
        acc = acc + ud * jnp.tanh(truth * wd + bd)
    yext_ref[0:BT, :] = acc
    yext_ref[BT:BT + 1, :] = jnp.ones((1, N), jnp.float32)


def _main_body(nv1t_ref, nv2t_ref, nv1f_ref, nv2f_ref, yext_ref,
               tr_ref, mk_ref, wm_ref, bg_ref, wct_ref, bc_ref, bm_ref,
               out_ref):
    i = pl.program_id(0)
    nv1t = nv1t_ref[...]
    nv2t = nv2t_ref[...]
    raw = _dg(nv1t, nv2f_ref[...], (1, 1)) - _dg(nv2t, nv1f_ref[...], (1, 1))
    a = jnp.maximum(jnp.tanh(ALPHA * raw), 0.0)
    big = 8192.0
    thresh = -(big * K - big / 2 - 2048.0)
    t = jnp.full((ROWS, 1), jnp.inf, jnp.float32)
    for _ in range(K):
        m2 = jnp.where(a >= t, -big, a)
        ssum = jnp.sum(m2, axis=1, keepdims=True)
        newt = jnp.max(m2, axis=1, keepdims=True)
        t = jnp.where(ssum <= thresh, t, newt)
    gt = a > t
    m3 = jnp.where(gt, -big, a)
    g = jnp.ceil(-jnp.sum(m3, axis=1, keepdims=True) * (1.0 / big))
    eq = a == t
    ps = jnp.where(eq, 1.0, 0.0)
    sh = 1
    while sh < N:
        ps = ps + jnp.concatenate(
            [jnp.zeros((ROWS, sh), jnp.float32), ps[:, :N - sh]], axis=1)
        sh *= 2
    keep = gt | (eq & (ps < (K + 1.0 - g)))
    adjm = jnp.where(keep, a, 0.0)
    z_ext = _dg(yext_ref[...], adjm, (1, 1))
    znorm = z_ext[0:BT, :] / (z_ext[BT:BT + 1, :] + 1e-6)
    c = jnp.sum(bg_ref[...] * wct_ref[...], keepdims=True) + bc_ref[...]
    x2 = jnp.tanh(znorm + c)
    mk = mk_ref[...]
    x2 = tr_ref[...] * mk + x2 * (1.0 - mk)
    contrib = _dg(x2, wm_ref[...], (1, 0))

    @pl.when(i == 0)
    def _():
        out_ref[...] = contrib + bm_ref[...]

    @pl.when(i != 0)
    def _():
        out_ref[...] = out_ref[...] + contrib


def kernel(truth, mask, emb1, emb2, lin1, lin2, W_d, b_d, W_g, b_g, W_c, b_c,
           W_m, b_m):
    B, T, _ = truth.shape
    truth2 = truth.reshape(BT, N)
    mask2 = mask.reshape(BT, N)

    nv1, nv2, yext = pl.pallas_call(
        _preamble_body,
        out_shape=[
            jax.ShapeDtypeStruct((N, D), jnp.float32),
            jax.ShapeDtypeStruct((N, D), jnp.float32),
            jax.ShapeDtypeStruct((BT + 1, N), jnp.float32),
        ],
    )(emb1, lin1, emb2, lin2, truth2,
      W_d.reshape(1, D), b_d.reshape(1, D), W_g, W_c)

    grid = (N // ROWS,)
    out = pl.pallas_call(
        _main_body,
        grid=grid,
        in_specs=[
            pl.BlockSpec((ROWS, D), lambda i: (i, 0)),
            pl.BlockSpec((ROWS, D), lambda i: (i, 0)),
            pl.BlockSpec((N, D), lambda i: (0, 0)),
            pl.BlockSpec((N, D), lambda i: (0, 0)),
            pl.BlockSpec((BT + 1, N), lambda i: (0, 0)),
            pl.BlockSpec((BT, ROWS), lambda i: (0, i)),
            pl.BlockSpec((BT, ROWS), lambda i: (0, i)),
            pl.BlockSpec((ROWS, OUT), lambda i: (i, 0)),
            pl.BlockSpec((1, D), lambda i: (0, 0)),
            pl.BlockSpec((1, D), lambda i: (0, 0)),
            pl.BlockSpec((1, 1), lambda i: (0, 0)),
            pl.BlockSpec((1, OUT), lambda i: (0, 0)),
        ],
        out_specs=pl.BlockSpec((BT, OUT), lambda i: (0, 0)),
        out_shape=jax.ShapeDtypeStruct((BT, OUT), jnp.float32),
    )(nv1, nv2, nv1, nv2, yext, truth2, mask2, W_m,
      b_g.reshape(1, D), W_c.reshape(1, D), b_c.reshape(1, 1),
      b_m.reshape(1, OUT))

    return out.reshape(B, T, OUT)
